# speedup vs baseline: 2.9599x; 2.9599x over previous
import functools

import jax
import jax.numpy as jnp
from jax import lax
from jax.experimental import pallas as pl
from jax.experimental.pallas import tpu as pltpu
from jax.experimental.pallas import tpu_sc as plsc

D = 128
B = 4096 * 50
NC, NS = 2, 16
NW = NC * NS
BPW = B // NW
C = 128
J = BPW // C

_mesh = plsc.VectorSubcoreMesh(core_axis_name="c", subcore_axis_name="s")


@functools.partial(
    pl.kernel,
    out_type=jax.ShapeDtypeStruct((B, D), jnp.float32),
    mesh=_mesh,
    scratch_types=[
        pltpu.VMEM((J, C), jnp.int32),
        pltpu.VMEM((C, D), jnp.float32),
        pltpu.SemaphoreType.DMA,
    ],
)
def _gather_kernel(idx_hbm, table_hbm, out_hbm, idx_v, rows_v, sem):
    wid = lax.axis_index("s") * NC + lax.axis_index("c")
    pltpu.sync_copy(idx_hbm.at[wid], idx_v)

    def body(j, carry):
        pltpu.async_copy(table_hbm.at[idx_v.at[j]], rows_v, sem).wait()
        pltpu.sync_copy(rows_v, out_hbm.at[pl.ds(wid * BPW + j * C, C)])
        return carry

    lax.fori_loop(0, J, body, 0)


def kernel(token_ids, weight):
    idx = token_ids.reshape(NW, J, C).astype(jnp.int32)
    out = _gather_kernel(idx, weight)
    return out.reshape(4096, 50, D)


# speedup vs baseline: 3.3291x; 1.1247x over previous
import functools

import jax
import jax.numpy as jnp
from jax import lax
from jax.experimental import pallas as pl
from jax.experimental.pallas import tpu as pltpu
from jax.experimental.pallas import tpu_sc as plsc

D = 128
B = 4096 * 50
NC, NS = 2, 16
NW = NC * NS
BPW = B // NW
C = 128
J = BPW // C
NBUF = 3

_mesh = plsc.VectorSubcoreMesh(core_axis_name="c", subcore_axis_name="s")


@functools.partial(
    pl.kernel,
    out_type=jax.ShapeDtypeStruct((B, D), jnp.float32),
    mesh=_mesh,
    scratch_types=[
        pltpu.VMEM((J, C), jnp.int32),
        pltpu.VMEM((C, D), jnp.float32),
        pltpu.VMEM((C, D), jnp.float32),
        pltpu.VMEM((C, D), jnp.float32),
        pltpu.SemaphoreType.DMA,
        pltpu.SemaphoreType.DMA,
        pltpu.SemaphoreType.DMA,
        pltpu.SemaphoreType.DMA,
        pltpu.SemaphoreType.DMA,
        pltpu.SemaphoreType.DMA,
    ],
)
def _gather_kernel(idx_hbm, table_hbm, out_hbm,
                   idx_v, r0, r1, r2, g0, g1, g2, o0, o1, o2):
    rows = (r0, r1, r2)
    sg = (g0, g1, g2)
    so = (o0, o1, o2)
    wid = lax.axis_index("s") * NC + lax.axis_index("c")
    base = wid * BPW
    pltpu.sync_copy(idx_hbm.at[wid], idx_v)

    def gather_start(j, b):
        pltpu.make_async_copy(table_hbm.at[idx_v.at[j]], rows[b], sg[b]).start()

    def gather_wait(b):
        pltpu.make_async_copy(table_hbm.at[idx_v.at[0]], rows[b], sg[b]).wait()

    def out_start(j, b):
        pltpu.make_async_copy(rows[b], out_hbm.at[pl.ds(base + j * C, C)],
                              so[b]).start()

    def out_wait(j, b):
        pltpu.make_async_copy(rows[b], out_hbm.at[pl.ds(base + j * C, C)],
                              so[b]).wait()

    gather_start(0, 0)
    gather_start(1, 1)
    gather_start(2, 2)
    gather_wait(0)
    out_start(0, 0)
    out_wait(0, 0)
    gather_start(3, 0)
    gather_wait(1)
    out_start(1, 1)
    out_wait(1, 1)
    gather_start(4, 1)
    gather_wait(2)
    out_start(2, 2)

    def body(g, carry):
        for b in range(NBUF):
            j = g * NBUF + b
            bn = (b + 2) % NBUF
            out_wait(j - 1, bn)
            gather_start(j + 2, bn)
            gather_wait(b)
            out_start(j, b)
        return carry

    lax.fori_loop(1, (J - 2) // NBUF, body, 0)

    out_wait(J - 3, (J - 3) % NBUF)
    gather_wait((J - 2) % NBUF)
    out_start(J - 2, (J - 2) % NBUF)
    gather_wait((J - 1) % NBUF)
    out_start(J - 1, (J - 1) % NBUF)
    out_wait(J - 2, (J - 2) % NBUF)
    out_wait(J - 1, (J - 1) % NBUF)


def kernel(token_ids, weight):
    idx = token_ids.reshape(NW, J, C).astype(jnp.int32)
    out = _gather_kernel(idx, weight)
    return out.reshape(4096, 50, D)


# speedup vs baseline: 3.3407x; 1.0035x over previous
import functools

import jax
import jax.numpy as jnp
from jax import lax
from jax.experimental import pallas as pl
from jax.experimental.pallas import tpu as pltpu
from jax.experimental.pallas import tpu_sc as plsc

D = 128
B = 4096 * 50
NC, NS = 2, 16
NW = NC * NS
BPW = B // NW
C = 128
J = BPW // C
K = 3
NFULL = J // K
RK = J - NFULL * K

_mesh = plsc.VectorSubcoreMesh(core_axis_name="c", subcore_axis_name="s")


@functools.partial(
    pl.kernel,
    out_type=jax.ShapeDtypeStruct((B, D), jnp.float32),
    mesh=_mesh,
    scratch_types=[
        pltpu.VMEM((J, C), jnp.int32),
        pltpu.VMEM((K * C, D), jnp.float32),
        pltpu.VMEM((K * C, D), jnp.float32),
        pltpu.SemaphoreType.DMA,
        pltpu.SemaphoreType.DMA,
        pltpu.SemaphoreType.DMA,
        pltpu.SemaphoreType.DMA,
    ],
)
def _gather_kernel(idx_hbm, table_hbm, out_hbm,
                   idx_v, r0, r1, g0, g1, o0, o1):
    rows = (r0, r1)
    sg = (g0, g1)
    so = (o0, o1)
    wid = lax.axis_index("s") * NC + lax.axis_index("c")
    base = wid * BPW
    pltpu.sync_copy(idx_hbm.at[wid], idx_v)

    def gather_start(r, b, k=K):
        for i in range(k):
            pltpu.make_async_copy(table_hbm.at[idx_v.at[r * K + i]],
                                  rows[b].at[pl.ds(i * C, C)], sg[b]).start()

    def gather_wait(b, k=K):
        pltpu.make_async_copy(table_hbm.at[idx_v.at[0]],
                              rows[b].at[pl.ds(0, k * C)], sg[b]).wait()

    def out_start(r, b, k=K):
        pltpu.make_async_copy(rows[b].at[pl.ds(0, k * C)],
                              out_hbm.at[pl.ds(base + r * K * C, k * C)],
                              so[b]).start()

    def out_wait(r, b, k=K):
        pltpu.make_async_copy(rows[b].at[pl.ds(0, k * C)],
                              out_hbm.at[pl.ds(base + r * K * C, k * C)],
                              so[b]).wait()

    gather_start(0, 0)
    gather_start(1, 1)
    gather_wait(0)
    out_start(0, 0)

    def body(g, carry):
        for b in range(2):
            r = 1 + g * 2 + b
            out_wait(r - 1, b)
            gather_start(r + 1, b)
            gather_wait(1 - b)
            out_start(r, 1 - b)
        return carry

    lax.fori_loop(0, (NFULL - 2) // 2, body, 0)

    out_wait(NFULL - 2, 0)
    gather_start(NFULL, 0, k=RK)
    gather_wait(1)
    out_start(NFULL - 1, 1)
    out_wait(NFULL - 1, 1)
    gather_wait(0, k=RK)
    out_start(NFULL, 0, k=RK)
    out_wait(NFULL, 0, k=RK)


def kernel(token_ids, weight):
    idx = token_ids.reshape(NW, J, C).astype(jnp.int32)
    out = _gather_kernel(idx, weight)
    return out.reshape(4096, 50, D)
